# baseline (speedup 1.0000x reference)
import functools

import jax
import jax.numpy as jnp
from jax.experimental import pallas as pl
from jax.experimental.pallas import tpu as pltpu

_NUM_NODES = 10000
_NUM_EDGES = 160000
_HID = 64


def _sparsify_kernel(ew_ref, ni_ref, ew_out_ref, ni_out_ref):
    ew = ew_ref[...]
    ew_out_ref[...] = jnp.where(ew < 0.2, jnp.zeros_like(ew), jnp.minimum(ew, 1.0))
    ni_out_ref[...] = jnp.clip(ni_ref[...], 0.0, 1.0)


def _scale_matmul_kernel(x_ref, ni_ref, w_ref, out_ref):
    xs = x_ref[...] * ni_ref[...]
    out_ref[...] = jnp.dot(xs, w_ref[...], preferred_element_type=jnp.float32)


def _readout_kernel(x_ref, wn_ref, bn_ref, wc_ref, bc_ref, out_ref, acc_ref):
    k = pl.program_id(0)

    @pl.when(k == 0)
    def _():
        acc_ref[...] = jnp.zeros_like(acc_ref)

    acc_ref[...] += jnp.dot(
        x_ref[...], wn_ref[...], preferred_element_type=jnp.float32
    )

    @pl.when(k == pl.num_programs(0) - 1)
    def _():
        hid = acc_ref[...] + bn_ref[...]
        out_ref[...] = (
            jnp.dot(hid, wc_ref[...], preferred_element_type=jnp.float32)
            + bc_ref[...]
        )


@jax.jit
def kernel(x, edge_index, batch, edge_weights, node_importance, W1, b1, Wn, bn, Wc, bc):
    total_nodes = x.shape[0]
    total_edges = edge_index.shape[1]
    bs_e = total_edges // _NUM_EDGES
    bs = batch.shape[0] // _NUM_NODES

    ew_2d = edge_weights.reshape(_NUM_EDGES // 128, 128)
    ni_2d = node_importance.reshape(1, _NUM_NODES)
    ew_s2, ni_c2 = pl.pallas_call(
        _sparsify_kernel,
        out_shape=(
            jax.ShapeDtypeStruct(ew_2d.shape, jnp.float32),
            jax.ShapeDtypeStruct(ni_2d.shape, jnp.float32),
        ),
    )(ew_2d, ni_2d)
    ew = ew_s2.reshape(-1)
    ni_clamped = ni_c2.reshape(-1)

    ni_exp = jnp.tile(node_importance, bs_e).reshape(total_nodes, 1)
    blk = 4000
    hW = pl.pallas_call(
        _scale_matmul_kernel,
        grid=(total_nodes // blk,),
        in_specs=[
            pl.BlockSpec((blk, x.shape[1]), lambda i: (i, 0)),
            pl.BlockSpec((blk, 1), lambda i: (i, 0)),
            pl.BlockSpec(W1.shape, lambda i: (0, 0)),
        ],
        out_specs=pl.BlockSpec((blk, _HID), lambda i: (i, 0)),
        out_shape=jax.ShapeDtypeStruct((total_nodes, _HID), jnp.float32),
    )(x, ni_exp, W1)

    row = edge_index[0]
    col = edge_index[1]
    ew_exp = jnp.tile(ew, bs_e)
    deg = jax.ops.segment_sum(ew_exp, col, num_segments=total_nodes) + 1.0
    dis = deg ** -0.5
    norm = dis[row] * ew_exp * dis[col]
    msg = hW[row] * norm[:, None]
    agg = jax.ops.segment_sum(msg, col, num_segments=total_nodes)
    agg = agg + hW * (dis * dis)[:, None]
    h = jax.nn.relu(agg + b1)

    starts = jnp.searchsorted(batch, jnp.arange(bs + 1, dtype=batch.dtype))
    cnts = starts[1:] - starts[:-1]
    hflat = jnp.concatenate(
        [h.reshape(-1), jnp.zeros((_NUM_NODES * _HID,), jnp.float32)]
    )
    seg = _NUM_NODES * _HID
    X = jnp.stack(
        [
            jax.lax.dynamic_slice(hflat, (starts[i] * _HID,), (seg,))
            for i in range(bs)
        ]
    )
    lane = jax.lax.iota(jnp.int32, seg)
    X = jnp.where(lane[None, :] < cnts[:, None] * _HID, X, 0.0)

    kb = 6400
    nsteps = seg // kb
    out = pl.pallas_call(
        _readout_kernel,
        grid=(nsteps,),
        in_specs=[
            pl.BlockSpec((bs, kb), lambda k: (0, k)),
            pl.BlockSpec((kb, Wn.shape[1]), lambda k: (k, 0)),
            pl.BlockSpec((1, bn.shape[0]), lambda k: (0, 0)),
            pl.BlockSpec(Wc.shape, lambda k: (0, 0)),
            pl.BlockSpec((1, bc.shape[0]), lambda k: (0, 0)),
        ],
        out_specs=pl.BlockSpec((bs, bc.shape[0]), lambda k: (0, 0)),
        out_shape=jax.ShapeDtypeStruct((bs, bc.shape[0]), jnp.float32),
        scratch_shapes=[pltpu.VMEM((bs, Wn.shape[1]), jnp.float32)],
    )(X, Wn, bn.reshape(1, -1), Wc, bc.reshape(1, -1))

    return (out, ew, ni_clamped)


# speedup vs baseline: 1.0052x; 1.0052x over previous
import functools

import jax
import jax.numpy as jnp
from jax.experimental import pallas as pl
from jax.experimental.pallas import tpu as pltpu

_NUM_NODES = 10000
_NUM_EDGES = 160000
_HID = 64


def _sparsify_kernel(ew_ref, ni_ref, ew_out_ref, ni_out_ref):
    ew = ew_ref[...]
    ew_out_ref[...] = jnp.where(ew < 0.2, jnp.zeros_like(ew), jnp.minimum(ew, 1.0))
    ni_out_ref[...] = jnp.clip(ni_ref[...], 0.0, 1.0)


def _scale_matmul_kernel(x_ref, ni_ref, w_ref, out_ref):
    xs = x_ref[...] * ni_ref[...]
    out_ref[...] = jnp.dot(xs, w_ref[...], preferred_element_type=jnp.float32)


def _readout_kernel(x_ref, wn_ref, bn_ref, wc_ref, bc_ref, out_ref, acc_ref):
    k = pl.program_id(0)

    @pl.when(k == 0)
    def _():
        acc_ref[...] = jnp.zeros_like(acc_ref)

    acc_ref[...] += jnp.dot(
        x_ref[...], wn_ref[...], preferred_element_type=jnp.float32
    )

    @pl.when(k == pl.num_programs(0) - 1)
    def _():
        hid = acc_ref[...] + bn_ref[...]
        out_ref[...] = (
            jnp.dot(hid, wc_ref[...], preferred_element_type=jnp.float32)
            + bc_ref[...]
        )


@jax.jit
def kernel(x, edge_index, batch, edge_weights, node_importance, W1, b1, Wn, bn, Wc, bc):
    total_nodes = x.shape[0]
    total_edges = edge_index.shape[1]
    bs_e = total_edges // _NUM_EDGES
    bs = batch.shape[0] // _NUM_NODES

    ew_2d = edge_weights.reshape(_NUM_EDGES // 128, 128)
    ni_2d = node_importance.reshape(1, _NUM_NODES)
    ew_s2, ni_c2 = pl.pallas_call(
        _sparsify_kernel,
        out_shape=(
            jax.ShapeDtypeStruct(ew_2d.shape, jnp.float32),
            jax.ShapeDtypeStruct(ni_2d.shape, jnp.float32),
        ),
    )(ew_2d, ni_2d)
    ew = ew_s2.reshape(-1)
    ni_clamped = ni_c2.reshape(-1)

    ni_exp = jnp.tile(node_importance, bs_e).reshape(total_nodes, 1)
    blk = 4000
    hW = pl.pallas_call(
        _scale_matmul_kernel,
        grid=(total_nodes // blk,),
        in_specs=[
            pl.BlockSpec((blk, x.shape[1]), lambda i: (i, 0)),
            pl.BlockSpec((blk, 1), lambda i: (i, 0)),
            pl.BlockSpec(W1.shape, lambda i: (0, 0)),
        ],
        out_specs=pl.BlockSpec((blk, _HID), lambda i: (i, 0)),
        out_shape=jax.ShapeDtypeStruct((total_nodes, _HID), jnp.float32),
    )(x, ni_exp, W1)

    row = edge_index[0]
    col = edge_index[1]
    ew_exp = jnp.tile(ew, bs_e)
    deg = jax.ops.segment_sum(ew_exp, col, num_segments=total_nodes) + 1.0
    dis = deg ** -0.5
    norm = dis[row] * ew_exp * dis[col]
    msg = hW[row] * norm[:, None]
    agg = jax.ops.segment_sum(msg, col, num_segments=total_nodes)
    agg = agg + hW * (dis * dis)[:, None]
    h = jax.nn.relu(agg + b1)

    starts = jnp.searchsorted(batch, jnp.arange(bs + 1, dtype=batch.dtype))
    cnts = starts[1:] - starts[:-1]
    hflat = jnp.concatenate(
        [h.reshape(-1), jnp.zeros((_NUM_NODES * _HID,), jnp.float32)]
    )
    seg = _NUM_NODES * _HID
    X = jnp.stack(
        [
            jax.lax.dynamic_slice(hflat, (starts[i] * _HID,), (seg,))
            for i in range(bs)
        ]
    )
    lane = jax.lax.iota(jnp.int32, seg)
    X = jnp.where(lane[None, :] < cnts[:, None] * _HID, X, 0.0)

    out = (X @ Wn + bn) @ Wc + bc

    return (out, ew, ni_clamped)


# speedup vs baseline: 3.8415x; 3.8217x over previous
import jax
import jax.numpy as jnp
from jax.experimental import pallas as pl
from jax.experimental.pallas import tpu as pltpu

_NUM_NODES = 10000
_NUM_EDGES = 160000
_HID = 64


def _sparsify_kernel(ew_ref, ni_ref, ew_out_ref, ni_out_ref):
    ew = ew_ref[...]
    ew_out_ref[...] = jnp.where(ew < 0.2, jnp.zeros_like(ew), jnp.minimum(ew, 1.0))
    ni_out_ref[...] = jnp.clip(ni_ref[...], 0.0, 1.0)


def _scale_matmul_kernel(x_ref, ni_ref, w_ref, out_ref):
    xs = x_ref[...] * ni_ref[...]
    out_ref[...] = jnp.dot(xs, w_ref[...], preferred_element_type=jnp.float32)


def _readout_kernel(x_ref, wn_ref, bn_ref, wc_ref, bc_ref, out_ref, acc_ref):
    k = pl.program_id(0)

    @pl.when(k == 0)
    def _():
        acc_ref[...] = jnp.zeros_like(acc_ref)

    acc_ref[...] += jnp.dot(
        x_ref[...], wn_ref[...], preferred_element_type=jnp.float32
    )

    @pl.when(k == pl.num_programs(0) - 1)
    def _():
        hid = acc_ref[...] + bn_ref[...]
        out_ref[...] = (
            jnp.dot(hid, wc_ref[...], preferred_element_type=jnp.float32)
            + bc_ref[...]
        )


@jax.jit
def kernel(x, edge_index, batch, edge_weights, node_importance, W1, b1, Wn, bn, Wc, bc):
    total_nodes = x.shape[0]
    total_edges = edge_index.shape[1]
    bs_e = total_edges // _NUM_EDGES
    bs = batch.shape[0] // _NUM_NODES

    ew_2d = edge_weights.reshape(_NUM_EDGES // 128, 128)
    ni_2d = node_importance.reshape(1, _NUM_NODES)
    ew_s2, ni_c2 = pl.pallas_call(
        _sparsify_kernel,
        out_shape=(
            jax.ShapeDtypeStruct(ew_2d.shape, jnp.float32),
            jax.ShapeDtypeStruct(ni_2d.shape, jnp.float32),
        ),
    )(ew_2d, ni_2d)
    ew = ew_s2.reshape(-1)
    ni_clamped = ni_c2.reshape(-1)

    ni_exp = jnp.tile(node_importance, bs_e).reshape(total_nodes, 1)
    blk = 4000
    hW = pl.pallas_call(
        _scale_matmul_kernel,
        grid=(total_nodes // blk,),
        in_specs=[
            pl.BlockSpec((blk, x.shape[1]), lambda i: (i, 0)),
            pl.BlockSpec((blk, 1), lambda i: (i, 0)),
            pl.BlockSpec(W1.shape, lambda i: (0, 0)),
        ],
        out_specs=pl.BlockSpec((blk, _HID), lambda i: (i, 0)),
        out_shape=jax.ShapeDtypeStruct((total_nodes, _HID), jnp.float32),
    )(x, ni_exp, W1)

    row = edge_index[0]
    col = edge_index[1]
    ew_exp = jnp.tile(ew, bs_e)
    deg = jax.ops.segment_sum(ew_exp, col, num_segments=total_nodes) + 1.0
    dis = deg ** -0.5
    hs = hW * dis[:, None]
    msg = hs[row] * ew_exp[:, None]
    agg = jax.ops.segment_sum(msg, col, num_segments=total_nodes)
    agg = (agg + hs) * dis[:, None]
    h = jax.nn.relu(agg + b1)

    starts = jnp.searchsorted(batch, jnp.arange(bs + 1, dtype=batch.dtype))
    rank = jnp.arange(total_nodes, dtype=jnp.int32) - starts[batch]
    dest = jnp.where(
        rank < _NUM_NODES, batch * _NUM_NODES + rank, bs * _NUM_NODES
    )
    X2 = jnp.zeros((bs * _NUM_NODES, _HID), jnp.float32).at[dest].set(
        h, mode="drop"
    )
    seg = _NUM_NODES * _HID
    X = X2.reshape(bs, seg)

    kb = 6400
    nsteps = seg // kb
    out = pl.pallas_call(
        _readout_kernel,
        grid=(nsteps,),
        in_specs=[
            pl.BlockSpec((bs, kb), lambda k: (0, k)),
            pl.BlockSpec((kb, Wn.shape[1]), lambda k: (k, 0)),
            pl.BlockSpec((1, bn.shape[0]), lambda k: (0, 0)),
            pl.BlockSpec(Wc.shape, lambda k: (0, 0)),
            pl.BlockSpec((1, bc.shape[0]), lambda k: (0, 0)),
        ],
        out_specs=pl.BlockSpec((bs, bc.shape[0]), lambda k: (0, 0)),
        out_shape=jax.ShapeDtypeStruct((bs, bc.shape[0]), jnp.float32),
        scratch_shapes=[pltpu.VMEM((bs, Wn.shape[1]), jnp.float32)],
    )(X, Wn, bn.reshape(1, -1), Wc, bc.reshape(1, -1))

    return (out, ew, ni_clamped)
